# baseline (speedup 1.0000x reference)
import jax
import jax.numpy as jnp
from jax import lax
from jax.experimental import pallas as pl
from jax.experimental.pallas import tpu as pltpu
from jax.experimental.pallas import tpu_sc as plsc

N_NODES = 100000
D = 128
B = 10000
S = 16
L = 16
NC, NS = 2, 16
NW = NC * NS
CN = 8
CPW = 40
NB = 5
NPW = CPW * CN
NG_LAST = 2
HROWS = NPW // 2


def _body(table_hbm, idx_hbm, out_hbm, idx_raw, idx_v, rows_v, out_b,
          gsem, osem):
    wid = lax.axis_index("s") * NC + lax.axis_index("c")
    last = wid == NW - 1
    nbase = pl.multiple_of(jnp.where(last, B - NPW, wid * NPW), 8)
    ls = jnp.where(last, CPW - NG_LAST * NB, 0)
    ng = jnp.where(last, NG_LAST, CPW // NB)
    obase = pl.multiple_of(jnp.where(last, B - NG_LAST * NB * CN,
                                     wid * NPW), 8)

    for h in range(2):
        pltpu.sync_copy(
            idx_hbm.at[pl.ds(pl.multiple_of(nbase + h * HROWS, 8), HROWS)],
            idx_raw)

        def repack_step(c, carry, h=h):
            for n in range(CN):
                idx_v[h * (CPW // 2) + c, pl.ds(n * S, S)] = \
                    idx_raw[c * CN + n, :]
            return carry

        lax.fori_loop(0, CPW // 2, repack_step, 0)

    def issue(c, b):
        pltpu.async_copy(table_hbm.at[idx_v.at[c]], rows_v.at[b], gsem)

    def drain(b):
        pltpu.make_async_copy(table_hbm.at[idx_v.at[0]], rows_v.at[b],
                              gsem).wait()

    def owait(b):
        pltpu.make_async_copy(out_b.at[b],
                              out_hbm.at[pl.ds(0, CN)], osem).wait()

    for b in range(NB):
        issue(ls + b, b)

    inv = jnp.full((L,), 1.0 / S, dtype=jnp.float32)

    def group_step(g, carry):
        for b in range(NB):
            cl = g * NB + b
            drain(b)

            @pl.when(g > 0)
            def _():
                owait(b)

            def node_step(node, carry2):
                base = node * S
                acc = [rows_v[b, base, pl.ds(j * L, L)]
                       for j in range(D // L)]
                for s in range(1, S):
                    for j in range(D // L):
                        acc[j] = acc[j] + rows_v[b, base + s,
                                                 pl.ds(j * L, L)]
                for j in range(D // L):
                    out_b[b, node, pl.ds(j * L, L)] = acc[j] * inv
                return carry2

            lax.fori_loop(0, CN, node_step, 0)

            row = pl.multiple_of(obase + cl * CN, 8)
            pltpu.async_copy(out_b.at[b], out_hbm.at[pl.ds(row, CN)], osem)

            @pl.when(g < ng - 1)
            def _():
                issue(ls + cl + NB, b)
        return carry

    lax.fori_loop(0, ng, group_step, 0)

    for b in range(NB):
        owait(b)


@jax.jit
def _sc_mean_agg(table, idx):
    mesh = plsc.VectorSubcoreMesh(core_axis_name="c", subcore_axis_name="s")
    kfn = pl.kernel(
        _body,
        mesh=mesh,
        out_type=jax.ShapeDtypeStruct((B, D), jnp.float32),
        scratch_types=[
            pltpu.VMEM((HROWS, S), jnp.int32),
            pltpu.VMEM((CPW, CN * S), jnp.int32),
            pltpu.VMEM((NB, CN * S, D), jnp.float32),
            pltpu.VMEM((NB, CN, D), jnp.float32),
            pltpu.SemaphoreType.DMA,
            pltpu.SemaphoreType.DMA,
        ],
    )
    return kfn(table, idx)


def kernel(features_weight, nodes, neigh_idx):
    return _sc_mean_agg(features_weight, neigh_idx.astype(jnp.int32))


# speedup vs baseline: 1.0304x; 1.0304x over previous
import jax
import jax.numpy as jnp
from jax import lax
from jax.experimental import pallas as pl
from jax.experimental.pallas import tpu as pltpu
from jax.experimental.pallas import tpu_sc as plsc

N_NODES = 100000
D = 128
B = 10000
S = 16
L = 16
NC, NS = 2, 16
NW = NC * NS
CN = 8
CPW = 40
NB = 4
NPW = CPW * CN
NG_LAST = 3


def _body(table_hbm, idx_hbm, out_hbm, idx_raw, idx_v, rows_v, out_b,
          gsem, osem):
    wid = lax.axis_index("s") * NC + lax.axis_index("c")
    last = wid == NW - 1
    nbase = pl.multiple_of(jnp.where(last, B - NPW, wid * NPW), 8)
    ls = jnp.where(last, CPW - NG_LAST * NB, 0)
    ng = jnp.where(last, NG_LAST, CPW // NB)
    obase = pl.multiple_of(jnp.where(last, B - NG_LAST * NB * CN,
                                     wid * NPW), 8)

    pltpu.sync_copy(idx_hbm.at[pl.ds(nbase, NPW)], idx_raw)

    def repack_step(c, carry):
        for n in range(CN):
            idx_v[c, pl.ds(n * S, S)] = idx_raw[c * CN + n, :]
        return carry

    lax.fori_loop(0, CPW, repack_step, 0)

    def issue(c, b):
        pltpu.async_copy(table_hbm.at[idx_v.at[c]], rows_v.at[b], gsem)

    def drain(b):
        pltpu.make_async_copy(table_hbm.at[idx_v.at[0]], rows_v.at[b],
                              gsem).wait()

    def owait(b):
        pltpu.make_async_copy(out_b.at[b],
                              out_hbm.at[pl.ds(0, CN)], osem).wait()

    for b in range(NB):
        issue(ls + b, b)

    inv = jnp.full((L,), 1.0 / S, dtype=jnp.float32)

    def group_step(g, carry):
        for b in range(NB):
            cl = g * NB + b
            drain(b)

            @pl.when(g > 0)
            def _():
                owait(b)

            def node_step(node, carry2):
                base = node * S
                acc = [rows_v[b, base, pl.ds(j * L, L)]
                       for j in range(D // L)]
                for s in range(1, S):
                    for j in range(D // L):
                        acc[j] = acc[j] + rows_v[b, base + s,
                                                 pl.ds(j * L, L)]
                for j in range(D // L):
                    out_b[b, node, pl.ds(j * L, L)] = acc[j] * inv
                return carry2

            lax.fori_loop(0, CN, node_step, 0)

            row = pl.multiple_of(obase + cl * CN, 8)
            pltpu.async_copy(out_b.at[b], out_hbm.at[pl.ds(row, CN)], osem)

            @pl.when(g < ng - 1)
            def _():
                issue(ls + cl + NB, b)
        return carry

    lax.fori_loop(0, ng, group_step, 0)

    for b in range(NB):
        owait(b)


@jax.jit
def _sc_mean_agg(table, idx):
    mesh = plsc.VectorSubcoreMesh(core_axis_name="c", subcore_axis_name="s")
    kfn = pl.kernel(
        _body,
        mesh=mesh,
        out_type=jax.ShapeDtypeStruct((B, D), jnp.float32),
        scratch_types=[
            pltpu.VMEM((NPW, S), jnp.int32),
            pltpu.VMEM((CPW, CN * S), jnp.int32),
            pltpu.VMEM((NB, CN * S, D), jnp.float32),
            pltpu.VMEM((NB, CN, D), jnp.float32),
            pltpu.SemaphoreType.DMA,
            pltpu.SemaphoreType.DMA,
        ],
    )
    return kfn(table, idx)


def kernel(features_weight, nodes, neigh_idx):
    return _sc_mean_agg(features_weight, neigh_idx.astype(jnp.int32))
